# baseline (speedup 1.0000x reference)
import functools
import math

import jax
import jax.numpy as jnp
from jax import lax
from jax.experimental import pallas as pl
from jax.experimental.pallas import tpu as pltpu
from jax.experimental.pallas import tpu_sc as plsc

DIM = 512
H = 16
K = 32
HD = DIM // H

_TQ = 1024
_TN = 256
_TA = 128
_SC_C = 64


def _qkv_body(x_ref, wq_ref, wk_ref, wv_ref, bq_ref, bk_ref, bv_ref,
              q_ref, kv_ref):
    xt = x_ref[...]
    q_ref[...] = jnp.dot(xt, wq_ref[...],
                         preferred_element_type=jnp.float32) + bq_ref[...]
    kf = jnp.dot(xt, wk_ref[...],
                 preferred_element_type=jnp.float32) + bk_ref[...]
    vf = jnp.dot(xt, wv_ref[...],
                 preferred_element_type=jnp.float32) + bv_ref[...]
    kv_ref[:, 0:DIM] = kf.astype(jnp.bfloat16)
    kv_ref[:, DIM:2 * DIM] = vf.astype(jnp.bfloat16)


def _qkv(xf, wqT, wkT, wvT, bq2, bk2, bv2):
    bn = xf.shape[0]
    grid = (bn // _TQ,)
    row = pl.BlockSpec((_TQ, DIM), lambda i: (i, 0))
    w = pl.BlockSpec((DIM, DIM), lambda i: (0, 0))
    b = pl.BlockSpec((1, DIM), lambda i: (0, 0))
    return pl.pallas_call(
        _qkv_body,
        grid=grid,
        in_specs=[row, w, w, w, b, b, b],
        out_specs=[row, pl.BlockSpec((_TQ, 2 * DIM), lambda i: (i, 0))],
        out_shape=[jax.ShapeDtypeStruct((bn, DIM), jnp.float32),
                   jax.ShapeDtypeStruct((bn, 2 * DIM), jnp.bfloat16)],
    )(xf, wqT, wkT, wvT, bq2, bk2, bv2)


def _knn_body(n, xyz_ref, xyzT_ref, idx_ref):
    bidx = pl.program_id(0)
    xt = xyz_ref[0]
    xa = xyzT_ref[0]
    x2t = jnp.sum(xt * xt, axis=1, keepdims=True)
    x2a = jnp.sum(xa * xa, axis=0, keepdims=True)
    cross = jax.lax.dot_general(
        xt.astype(jnp.bfloat16), xa.astype(jnp.bfloat16),
        (((1,), (0,)), ((), ())),
        preferred_element_type=jnp.float32)
    d2 = jnp.maximum(x2t + x2a - 2.0 * cross, 0.0)
    iota = lax.broadcasted_iota(jnp.int32, (_TN, n), 1)
    inf = jnp.float32(jnp.inf)
    cols = []
    for _ in range(K):
        m = jnp.min(d2, axis=1, keepdims=True)
        miota = jnp.where(d2 <= m, iota, n)
        am = jnp.min(miota, axis=1, keepdims=True)
        cols.append(am)
        d2 = jnp.where(iota == am, inf, d2)
    idx_ref[0] = jnp.concatenate(cols, axis=1) + bidx * n


def _knn(xyzp8, xyzT):
    bsz, n = xyzp8.shape[0], xyzp8.shape[1]
    grid = (bsz, n // _TN)
    return pl.pallas_call(
        functools.partial(_knn_body, n),
        grid=grid,
        in_specs=[
            pl.BlockSpec((1, _TN, 8), lambda bi, i: (bi, i, 0)),
            pl.BlockSpec((1, 8, n), lambda bi, i: (bi, 0, 0)),
        ],
        out_specs=pl.BlockSpec((1, _TN, K), lambda bi, i: (bi, i, 0)),
        out_shape=jax.ShapeDtypeStruct((bsz, n, K), jnp.int32),
    )(xyzp8, xyzT)


def _make_gather(n_idx):
    mesh = plsc.VectorSubcoreMesh(core_axis_name="c", subcore_axis_name="s")
    nw = 32
    per_w = n_idx // nw
    n_it = per_w // _SC_C
    n_pair = n_it // 2
    c = _SC_C

    @functools.partial(
        pl.kernel,
        out_type=(
            jax.ShapeDtypeStruct((n_idx, DIM), jnp.int32),
            jax.ShapeDtypeStruct((n_idx, 128), jnp.float32),
        ),
        mesh=mesh,
        scratch_types=[
            pltpu.VMEM((c,), jnp.int32),
            pltpu.VMEM((c,), jnp.int32),
            pltpu.VMEM((c, DIM), jnp.int32),
            pltpu.VMEM((c, DIM), jnp.int32),
            pltpu.VMEM((c, 128), jnp.float32),
            pltpu.VMEM((c, 128), jnp.float32),
            pltpu.SemaphoreType.DMA,
            pltpu.SemaphoreType.DMA,
            pltpu.SemaphoreType.DMA,
            pltpu.SemaphoreType.DMA,
        ],
    )
    def gather_k(kvtab, xtab, idx_hbm, kvnb, xnb,
                 idx0, idx1, rkv0, rkv1, rx0, rx1, skv0, skv1, sx0, sx1):
        wid = lax.axis_index("s") * 2 + lax.axis_index("c")
        base0 = wid * per_w

        def load_fire(chunk, idx_v, rkv, rx, skv, sx):
            pltpu.sync_copy(idx_hbm.at[pl.ds(base0 + chunk * c, c)], idx_v)
            pltpu.async_copy(kvtab.at[idx_v], rkv, skv)
            pltpu.async_copy(xtab.at[idx_v], rx, sx)

        def drain_write(chunk, idx_v, rkv, rx, skv, sx):
            pltpu.make_async_copy(kvtab.at[idx_v], rkv, skv).wait()
            pltpu.make_async_copy(xtab.at[idx_v], rx, sx).wait()
            pltpu.sync_copy(rkv, kvnb.at[pl.ds(base0 + chunk * c, c)])
            pltpu.sync_copy(rx, xnb.at[pl.ds(base0 + chunk * c, c)])

        def body(t, carry):
            c0 = 2 * t
            load_fire(c0 + 1, idx1, rkv1, rx1, skv1, sx1)
            drain_write(c0, idx0, rkv0, rx0, skv0, sx0)

            @pl.when(t + 1 < n_pair)
            def _():
                load_fire(c0 + 2, idx0, rkv0, rx0, skv0, sx0)

            drain_write(c0 + 1, idx1, rkv1, rx1, skv1, sx1)
            return carry

        load_fire(0, idx0, rkv0, rx0, skv0, sx0)
        lax.fori_loop(0, n_pair, body, 0)

    return gather_k


def _dot(a, b, prec=None):
    return jax.lax.dot_general(a, b, (((1,), (0,)), ((), ())),
                               precision=prec,
                               preferred_element_type=jnp.float32)


def _attn_body(q_ref, xyz_ref, kvnb_ref, nbx_ref,
               wb1_ref, bb1_ref, wb2_ref, bb2_ref, wo_ref, bo_ref,
               bd_ref, bdt_ref, y_ref):
    hi = None
    scale = jnp.float32(1.0 / math.sqrt(HD))
    q = q_ref[...] * scale
    xyz = xyz_ref[...]
    nbx = nbx_ref[...]
    rel = (xyz[:, None, :] - nbx).reshape(_TA * K, 128)
    h1 = jnp.maximum(_dot(rel, wb1_ref[...], hi) + bb1_ref[...], 0.0)
    bias2 = _dot(h1, wb2_ref[...], hi) + bb2_ref[...]

    kv = kvnb_ref[...]
    prod = (kv[:, :, 0:DIM] * q[:, None, :]).reshape(_TA * K, DIM)
    s2 = _dot(prod, bd_ref[...], hi) + bias2
    s3 = s2.reshape(_TA, K, H)
    m = jnp.max(s3, axis=1, keepdims=True)
    p = jnp.exp(s3 - m)
    l = jnp.sum(p, axis=1, keepdims=True)
    a2 = (p / l).reshape(_TA * K, H)
    aexp = _dot(a2, bdt_ref[...], hi).reshape(_TA, K, DIM)
    o = jnp.sum(aexp * kv[:, :, DIM:2 * DIM], axis=1)
    y_ref[...] = jnp.dot(o, wo_ref[...],
                         preferred_element_type=jnp.float32) + bo_ref[...]


def _attn(q, xyzp16, kvnb, xnb, wb1p, bb1_2, wb2T, bb2_2, woT, bo2, bd, bdt):
    bn = q.shape[0]
    grid = (bn // _TA,)
    row = pl.BlockSpec((_TA, DIM), lambda i: (i, 0))
    return pl.pallas_call(
        _attn_body,
        grid=grid,
        in_specs=[
            row,
            pl.BlockSpec((_TA, 128), lambda i: (i, 0)),
            pl.BlockSpec((_TA, K, 2 * DIM), lambda i: (i, 0, 0)),
            pl.BlockSpec((_TA, K, 128), lambda i: (i, 0, 0)),
            pl.BlockSpec((128, 32), lambda i: (0, 0)),
            pl.BlockSpec((1, 32), lambda i: (0, 0)),
            pl.BlockSpec((32, H), lambda i: (0, 0)),
            pl.BlockSpec((1, H), lambda i: (0, 0)),
            pl.BlockSpec((DIM, DIM), lambda i: (0, 0)),
            pl.BlockSpec((1, DIM), lambda i: (0, 0)),
            pl.BlockSpec((DIM, H), lambda i: (0, 0)),
            pl.BlockSpec((H, DIM), lambda i: (0, 0)),
        ],
        out_specs=row,
        out_shape=jax.ShapeDtypeStruct((bn, DIM), jnp.float32),
    )(q, xyzp16, kvnb, xnb, wb1p, bb1_2, wb2T, bb2_2, woT, bo2, bd, bdt)


def kernel(x, xyz, Wq, bq, Wk, bk, Wv, bv, Wo, bo, Wb1, bb1, Wb2, bb2):
    bsz, n, c = x.shape
    bn = bsz * n
    xf = x.reshape(bn, c)

    q, kvb = _qkv(xf, Wq.T, Wk.T, Wv.T, bq[None], bk[None], bv[None])

    xyzp8 = jnp.pad(xyz, ((0, 0), (0, 0), (0, 5)))
    xyzT = jnp.swapaxes(xyzp8, 1, 2)
    idx = _knn(xyzp8, xyzT)
    idx_flat = idx.reshape(bn * K)

    xyzp128 = jnp.pad(xyz.reshape(bn, 3), ((0, 0), (0, 125)))
    kv_i32 = jax.lax.bitcast_convert_type(kvb.reshape(bn, DIM, 2), jnp.int32)
    kvnb_i32, xnb = _make_gather(bn * K)(kv_i32, xyzp128, idx_flat)
    kvnb = jax.lax.bitcast_convert_type(kvnb_i32, jnp.bfloat16)

    wb1p = jnp.pad(Wb1.T, ((0, 125), (0, 0)))
    eye = jnp.eye(H, dtype=jnp.float32)
    bd = jnp.repeat(eye, HD, axis=0)
    y = _attn(q, xyzp128,
              kvnb.reshape(bn, K, 2 * DIM), xnb.reshape(bn, K, 128),
              wb1p, bb1[None], Wb2.T, bb2[None], Wo.T, bo[None],
              bd, bd.T)
    return y.reshape(bsz, n, c)


# speedup vs baseline: 4.9389x; 4.9389x over previous
import functools
import math

import jax
import jax.numpy as jnp
from jax import lax
from jax.experimental import pallas as pl
from jax.experimental.pallas import tpu as pltpu
from jax.experimental.pallas import tpu_sc as plsc

DIM = 512
H = 16
K = 32
HD = DIM // H

_TQ = 1024
_TN = 256
_TA = 128
_SC_C = 64


HDIM = DIM // 2


def _bf16_bits(x):
    b = jax.lax.bitcast_convert_type(x, jnp.uint32)
    r = b + jnp.uint32(0x7FFF) + ((b >> 16) & jnp.uint32(1))
    return r & jnp.uint32(0xFFFF0000)


def _pack_pair(lo_f32, hi_f32):
    w = _bf16_bits(hi_f32) | (_bf16_bits(lo_f32) >> 16)
    return jax.lax.bitcast_convert_type(w, jnp.int32)


def _qkv_body(x_ref, wq_ref, wk_ref, wv_ref, bq_ref, bk_ref, bv_ref,
              q_ref, kv_ref):
    xt = x_ref[...]
    q_ref[...] = jnp.dot(xt, wq_ref[...],
                         preferred_element_type=jnp.float32) + bq_ref[...]
    kf = jnp.dot(xt, wk_ref[...],
                 preferred_element_type=jnp.float32) + bk_ref[...]
    vf = jnp.dot(xt, wv_ref[...],
                 preferred_element_type=jnp.float32) + bv_ref[...]
    kv_ref[:, 0:HDIM] = _pack_pair(kf[:, 0:HDIM], kf[:, HDIM:DIM])
    kv_ref[:, HDIM:DIM] = _pack_pair(vf[:, 0:HDIM], vf[:, HDIM:DIM])


def _qkv(xf, wqT, wkT, wvT, bq2, bk2, bv2):
    bn = xf.shape[0]
    grid = (bn // _TQ,)
    row = pl.BlockSpec((_TQ, DIM), lambda i: (i, 0))
    w = pl.BlockSpec((DIM, DIM), lambda i: (0, 0))
    b = pl.BlockSpec((1, DIM), lambda i: (0, 0))
    return pl.pallas_call(
        _qkv_body,
        grid=grid,
        in_specs=[row, w, w, w, b, b, b],
        out_specs=[row, pl.BlockSpec((_TQ, DIM), lambda i: (i, 0))],
        out_shape=[jax.ShapeDtypeStruct((bn, DIM), jnp.float32),
                   jax.ShapeDtypeStruct((bn, DIM), jnp.int32)],
    )(xf, wqT, wkT, wvT, bq2, bk2, bv2)


def _knn_body(n, xyz_ref, xyzT_ref, idx_ref):
    bidx = pl.program_id(0)
    xt = xyz_ref[0]
    xa = xyzT_ref[0]
    x2t = jnp.sum(xt * xt, axis=1, keepdims=True)
    x2a = jnp.sum(xa * xa, axis=0, keepdims=True)
    cross = jax.lax.dot_general(
        xt.astype(jnp.bfloat16), xa.astype(jnp.bfloat16),
        (((1,), (0,)), ((), ())),
        preferred_element_type=jnp.float32)
    d2 = jnp.maximum(x2t + x2a - 2.0 * cross, 0.0)
    iota = lax.broadcasted_iota(jnp.int32, (_TN, n), 1)
    inf = jnp.float32(jnp.inf)
    cols = []
    for _ in range(K):
        m = jnp.min(d2, axis=1, keepdims=True)
        miota = jnp.where(d2 <= m, iota, n)
        am = jnp.min(miota, axis=1, keepdims=True)
        cols.append(am)
        d2 = jnp.where(iota == am, inf, d2)
    idx_ref[0] = jnp.concatenate(cols, axis=1) + bidx * n


def _knn(xyzp8, xyzT):
    bsz, n = xyzp8.shape[0], xyzp8.shape[1]
    grid = (bsz, n // _TN)
    return pl.pallas_call(
        functools.partial(_knn_body, n),
        grid=grid,
        in_specs=[
            pl.BlockSpec((1, _TN, 8), lambda bi, i: (bi, i, 0)),
            pl.BlockSpec((1, 8, n), lambda bi, i: (bi, 0, 0)),
        ],
        out_specs=pl.BlockSpec((1, _TN, K), lambda bi, i: (bi, i, 0)),
        out_shape=jax.ShapeDtypeStruct((bsz, n, K), jnp.int32),
    )(xyzp8, xyzT)


def _make_gather(n_idx):
    mesh = plsc.VectorSubcoreMesh(core_axis_name="c", subcore_axis_name="s")
    nw = 32
    per_w = n_idx // nw
    n_it = per_w // _SC_C
    n_pair = n_it // 2
    c = _SC_C

    @functools.partial(
        pl.kernel,
        out_type=(
            jax.ShapeDtypeStruct((n_idx, DIM), jnp.int32),
            jax.ShapeDtypeStruct((n_idx, 128), jnp.float32),
        ),
        mesh=mesh,
        scratch_types=[
            pltpu.VMEM((c,), jnp.int32),
            pltpu.VMEM((c,), jnp.int32),
            pltpu.VMEM((c, DIM), jnp.int32),
            pltpu.VMEM((c, DIM), jnp.int32),
            pltpu.VMEM((c, 128), jnp.float32),
            pltpu.VMEM((c, 128), jnp.float32),
            pltpu.SemaphoreType.DMA,
            pltpu.SemaphoreType.DMA,
            pltpu.SemaphoreType.DMA,
            pltpu.SemaphoreType.DMA,
        ],
    )
    def gather_k(kvtab, xtab, idx_hbm, kvnb, xnb,
                 idx0, idx1, rkv0, rkv1, rx0, rx1, skv0, skv1, sx0, sx1):
        wid = lax.axis_index("s") * 2 + lax.axis_index("c")
        base0 = wid * per_w

        def load_fire(chunk, idx_v, rkv, rx, skv, sx):
            pltpu.sync_copy(idx_hbm.at[pl.ds(base0 + chunk * c, c)], idx_v)
            pltpu.async_copy(kvtab.at[idx_v], rkv, skv)
            pltpu.async_copy(xtab.at[idx_v], rx, sx)

        def drain_write(chunk, idx_v, rkv, rx, skv, sx):
            pltpu.make_async_copy(kvtab.at[idx_v], rkv, skv).wait()
            pltpu.make_async_copy(xtab.at[idx_v], rx, sx).wait()
            pltpu.sync_copy(rkv, kvnb.at[pl.ds(base0 + chunk * c, c)])
            pltpu.sync_copy(rx, xnb.at[pl.ds(base0 + chunk * c, c)])

        def body(t, carry):
            c0 = 2 * t
            load_fire(c0 + 1, idx1, rkv1, rx1, skv1, sx1)
            drain_write(c0, idx0, rkv0, rx0, skv0, sx0)

            @pl.when(t + 1 < n_pair)
            def _():
                load_fire(c0 + 2, idx0, rkv0, rx0, skv0, sx0)

            drain_write(c0 + 1, idx1, rkv1, rx1, skv1, sx1)
            return carry

        load_fire(0, idx0, rkv0, rx0, skv0, sx0)
        lax.fori_loop(0, n_pair, body, 0)

    return gather_k


def _dot(a, b, prec=None):
    return jax.lax.dot_general(a, b, (((1,), (0,)), ((), ())),
                               precision=prec,
                               preferred_element_type=jnp.float32)


def _attn_body(q_ref, xyz_ref, kvnb_ref, nbx_ref,
               wb1_ref, bb1_ref, wb2_ref, bb2_ref, wo_ref, bo_ref,
               bd_ref, bdt_ref, y_ref):
    hi = None
    scale = jnp.float32(1.0 / math.sqrt(HD))
    q = q_ref[...] * scale
    xyz = xyz_ref[...]
    nbx = nbx_ref[...]
    rel = (xyz[:, None, :] - nbx).reshape(_TA * K, 128)
    h1 = jnp.maximum(_dot(rel, wb1_ref[...], hi) + bb1_ref[...], 0.0)
    bias2 = _dot(h1, wb2_ref[...], hi) + bb2_ref[...]

    kvw = jax.lax.bitcast_convert_type(kvnb_ref[...], jnp.uint32)
    kw = kvw[:, :, 0:HDIM]
    vw = kvw[:, :, HDIM:DIM]
    f32 = functools.partial(jax.lax.bitcast_convert_type,
                            new_dtype=jnp.float32)
    hi_mask = jnp.uint32(0xFFFF0000)
    k_lo, k_hi = f32(kw << 16), f32(kw & hi_mask)
    prod = jnp.concatenate(
        [k_lo * q[:, None, 0:HDIM], k_hi * q[:, None, HDIM:DIM]],
        axis=2).reshape(_TA * K, DIM)
    s2 = _dot(prod, bd_ref[...], hi) + bias2
    s3 = s2.reshape(_TA, K, H)
    m = jnp.max(s3, axis=1, keepdims=True)
    p = jnp.exp(s3 - m)
    l = jnp.sum(p, axis=1, keepdims=True)
    a2 = (p / l).reshape(_TA * K, H)
    aexp = _dot(a2, bdt_ref[...], hi).reshape(_TA, K, DIM)
    v_lo, v_hi = f32(vw << 16), f32(vw & hi_mask)
    o = jnp.concatenate(
        [jnp.sum(aexp[:, :, 0:HDIM] * v_lo, axis=1),
         jnp.sum(aexp[:, :, HDIM:DIM] * v_hi, axis=1)], axis=1)
    y_ref[...] = jnp.dot(o, wo_ref[...],
                         preferred_element_type=jnp.float32) + bo_ref[...]


def _attn(q, xyzp16, kvnb, xnb, wb1p, bb1_2, wb2T, bb2_2, woT, bo2, bd, bdt):
    bn = q.shape[0]
    grid = (bn // _TA,)
    row = pl.BlockSpec((_TA, DIM), lambda i: (i, 0))
    return pl.pallas_call(
        _attn_body,
        grid=grid,
        in_specs=[
            row,
            pl.BlockSpec((_TA, 128), lambda i: (i, 0)),
            pl.BlockSpec((_TA, K, DIM), lambda i: (i, 0, 0)),
            pl.BlockSpec((_TA, K, 128), lambda i: (i, 0, 0)),
            pl.BlockSpec((128, 32), lambda i: (0, 0)),
            pl.BlockSpec((1, 32), lambda i: (0, 0)),
            pl.BlockSpec((32, H), lambda i: (0, 0)),
            pl.BlockSpec((1, H), lambda i: (0, 0)),
            pl.BlockSpec((DIM, DIM), lambda i: (0, 0)),
            pl.BlockSpec((1, DIM), lambda i: (0, 0)),
            pl.BlockSpec((DIM, H), lambda i: (0, 0)),
            pl.BlockSpec((H, DIM), lambda i: (0, 0)),
        ],
        out_specs=row,
        out_shape=jax.ShapeDtypeStruct((bn, DIM), jnp.float32),
    )(q, xyzp16, kvnb, xnb, wb1p, bb1_2, wb2T, bb2_2, woT, bo2, bd, bdt)


def kernel(x, xyz, Wq, bq, Wk, bk, Wv, bv, Wo, bo, Wb1, bb1, Wb2, bb2):
    bsz, n, c = x.shape
    bn = bsz * n
    xf = x.reshape(bn, c)

    q, kvb = _qkv(xf, Wq.T, Wk.T, Wv.T, bq[None], bk[None], bv[None])

    xyzp8 = jnp.pad(xyz, ((0, 0), (0, 0), (0, 5)))
    xyzT = jnp.swapaxes(xyzp8, 1, 2)
    idx = _knn(xyzp8, xyzT)
    idx_flat = idx.reshape(bn * K)

    xyzp128 = jnp.pad(xyz.reshape(bn, 3), ((0, 0), (0, 125)))
    kvnb, xnb = _make_gather(bn * K)(kvb, xyzp128, idx_flat)

    wb1p = jnp.pad(Wb1.T, ((0, 125), (0, 0)))
    eye = jnp.eye(H, dtype=jnp.float32)
    bd = jnp.repeat(eye, HD, axis=0)
    y = _attn(q, xyzp128,
              kvnb.reshape(bn, K, DIM), xnb.reshape(bn, K, 128),
              wb1p, bb1[None], Wb2.T, bb2[None], Wo.T, bo[None],
              bd, bd.T)
    return y.reshape(bsz, n, c)


# speedup vs baseline: 5.3671x; 1.0867x over previous
import functools
import math

import jax
import jax.numpy as jnp
from jax import lax
from jax.experimental import pallas as pl
from jax.experimental.pallas import tpu as pltpu
from jax.experimental.pallas import tpu_sc as plsc

DIM = 512
H = 16
K = 32
HD = DIM // H

_TQ = 1024
_TN = 256
_TA = 128
_SC_C = 64


HDIM = DIM // 2


def _bf16_bits(x):
    b = jax.lax.bitcast_convert_type(x, jnp.uint32)
    r = b + jnp.uint32(0x7FFF) + ((b >> 16) & jnp.uint32(1))
    return r & jnp.uint32(0xFFFF0000)


def _pack_pair(lo_f32, hi_f32):
    w = _bf16_bits(hi_f32) | (_bf16_bits(lo_f32) >> 16)
    return jax.lax.bitcast_convert_type(w, jnp.int32)


def _qkv_body(x_ref, wq_ref, wk_ref, wv_ref, bq_ref, bk_ref, bv_ref,
              q_ref, kv_ref):
    xt = x_ref[...]
    q_ref[...] = jnp.dot(xt, wq_ref[...],
                         preferred_element_type=jnp.float32) + bq_ref[...]
    kf = jnp.dot(xt, wk_ref[...],
                 preferred_element_type=jnp.float32) + bk_ref[...]
    vf = jnp.dot(xt, wv_ref[...],
                 preferred_element_type=jnp.float32) + bv_ref[...]
    kv_ref[:, 0:HDIM] = _pack_pair(kf[:, 0:HDIM], kf[:, HDIM:DIM])
    kv_ref[:, HDIM:DIM] = _pack_pair(vf[:, 0:HDIM], vf[:, HDIM:DIM])


def _qkv(xf, wqT, wkT, wvT, bq2, bk2, bv2):
    bn = xf.shape[0]
    grid = (bn // _TQ,)
    row = pl.BlockSpec((_TQ, DIM), lambda i: (i, 0))
    w = pl.BlockSpec((DIM, DIM), lambda i: (0, 0))
    b = pl.BlockSpec((1, DIM), lambda i: (0, 0))
    return pl.pallas_call(
        _qkv_body,
        grid=grid,
        in_specs=[row, w, w, w, b, b, b],
        out_specs=[row, pl.BlockSpec((_TQ, DIM), lambda i: (i, 0))],
        out_shape=[jax.ShapeDtypeStruct((bn, DIM), jnp.float32),
                   jax.ShapeDtypeStruct((bn, DIM), jnp.int32)],
    )(xf, wqT, wkT, wvT, bq2, bk2, bv2)


def _knn_body(n, xyz_ref, xyzT_ref, idx_ref):
    bidx = pl.program_id(0)
    xt = xyz_ref[0]
    xa = xyzT_ref[0]
    x2t = jnp.sum(xt * xt, axis=1, keepdims=True)
    x2a = jnp.sum(xa * xa, axis=0, keepdims=True)
    cross = jax.lax.dot_general(
        xt.astype(jnp.bfloat16), xa.astype(jnp.bfloat16),
        (((1,), (0,)), ((), ())),
        preferred_element_type=jnp.float32)
    d2 = jnp.maximum(x2t + x2a - 2.0 * cross, 0.0)
    iota = lax.broadcasted_iota(jnp.int32, (_TN, n), 1)
    inf = jnp.float32(jnp.inf)
    cols = []
    for _ in range(K):
        am = jnp.argmin(d2, axis=1).astype(jnp.int32)[:, None]
        cols.append(am)
        d2 = jnp.where(iota == am, inf, d2)
    idx_ref[0] = jnp.concatenate(cols, axis=1) + bidx * n


def _knn(xyzp8, xyzT):
    bsz, n = xyzp8.shape[0], xyzp8.shape[1]
    grid = (bsz, n // _TN)
    return pl.pallas_call(
        functools.partial(_knn_body, n),
        grid=grid,
        in_specs=[
            pl.BlockSpec((1, _TN, 8), lambda bi, i: (bi, i, 0)),
            pl.BlockSpec((1, 8, n), lambda bi, i: (bi, 0, 0)),
        ],
        out_specs=pl.BlockSpec((1, _TN, K), lambda bi, i: (bi, i, 0)),
        out_shape=jax.ShapeDtypeStruct((bsz, n, K), jnp.int32),
    )(xyzp8, xyzT)


def _make_gather(n_idx):
    mesh = plsc.VectorSubcoreMesh(core_axis_name="c", subcore_axis_name="s")
    nw = 32
    per_w = n_idx // nw
    n_it = per_w // _SC_C
    n_pair = n_it // 2
    c = _SC_C

    @functools.partial(
        pl.kernel,
        out_type=(
            jax.ShapeDtypeStruct((n_idx, DIM), jnp.int32),
            jax.ShapeDtypeStruct((n_idx, 128), jnp.float32),
        ),
        mesh=mesh,
        scratch_types=[
            pltpu.VMEM((c,), jnp.int32),
            pltpu.VMEM((c,), jnp.int32),
            pltpu.VMEM((c, DIM), jnp.int32),
            pltpu.VMEM((c, DIM), jnp.int32),
            pltpu.VMEM((c, 128), jnp.float32),
            pltpu.VMEM((c, 128), jnp.float32),
            pltpu.SemaphoreType.DMA,
            pltpu.SemaphoreType.DMA,
            pltpu.SemaphoreType.DMA,
            pltpu.SemaphoreType.DMA,
        ],
    )
    def gather_k(kvtab, xtab, idx_hbm, kvnb, xnb,
                 idx0, idx1, rkv0, rkv1, rx0, rx1, skv0, skv1, sx0, sx1):
        wid = lax.axis_index("s") * 2 + lax.axis_index("c")
        base0 = wid * per_w

        def load_fire(chunk, idx_v, rkv, rx, skv, sx):
            pltpu.sync_copy(idx_hbm.at[pl.ds(base0 + chunk * c, c)], idx_v)
            pltpu.async_copy(kvtab.at[idx_v], rkv, skv)
            pltpu.async_copy(xtab.at[idx_v], rx, sx)

        def drain_write(chunk, idx_v, rkv, rx, skv, sx):
            pltpu.make_async_copy(kvtab.at[idx_v], rkv, skv).wait()
            pltpu.make_async_copy(xtab.at[idx_v], rx, sx).wait()
            pltpu.sync_copy(rkv, kvnb.at[pl.ds(base0 + chunk * c, c)])
            pltpu.sync_copy(rx, xnb.at[pl.ds(base0 + chunk * c, c)])

        def body(t, carry):
            c0 = 2 * t
            load_fire(c0 + 1, idx1, rkv1, rx1, skv1, sx1)
            drain_write(c0, idx0, rkv0, rx0, skv0, sx0)

            @pl.when(t + 1 < n_pair)
            def _():
                load_fire(c0 + 2, idx0, rkv0, rx0, skv0, sx0)

            drain_write(c0 + 1, idx1, rkv1, rx1, skv1, sx1)
            return carry

        load_fire(0, idx0, rkv0, rx0, skv0, sx0)
        lax.fori_loop(0, n_pair, body, 0)

    return gather_k


def _dot(a, b, prec=None):
    return jax.lax.dot_general(a, b, (((1,), (0,)), ((), ())),
                               precision=prec,
                               preferred_element_type=jnp.float32)


def _attn_body(q_ref, xyz_ref, kvnb_ref, nbx_ref,
               wb1_ref, bb1_ref, wb2_ref, bb2_ref, wo_ref, bo_ref,
               bd_ref, bdt_ref, y_ref):
    hi = None
    scale = jnp.float32(1.0 / math.sqrt(HD))
    q = q_ref[...] * scale
    xyz = xyz_ref[...]
    nbx = nbx_ref[...]
    rel = (xyz[:, None, :] - nbx).reshape(_TA * K, 128)
    h1 = jnp.maximum(_dot(rel, wb1_ref[...], hi) + bb1_ref[...], 0.0)
    bias2 = _dot(h1, wb2_ref[...], hi) + bb2_ref[...]

    kvw = jax.lax.bitcast_convert_type(kvnb_ref[...], jnp.uint32)
    kw = kvw[:, :, 0:HDIM]
    vw = kvw[:, :, HDIM:DIM]
    f32 = functools.partial(jax.lax.bitcast_convert_type,
                            new_dtype=jnp.float32)
    hi_mask = jnp.uint32(0xFFFF0000)
    k_lo, k_hi = f32(kw << 16), f32(kw & hi_mask)
    prod = jnp.concatenate(
        [k_lo * q[:, None, 0:HDIM], k_hi * q[:, None, HDIM:DIM]],
        axis=2).reshape(_TA * K, DIM)
    s2 = _dot(prod, bd_ref[...], hi) + bias2
    s3 = s2.reshape(_TA, K, H)
    m = jnp.max(s3, axis=1, keepdims=True)
    p = jnp.exp(s3 - m)
    l = jnp.sum(p, axis=1, keepdims=True)
    a2 = (p / l).reshape(_TA * K, H)
    aexp = _dot(a2, bdt_ref[...], hi).reshape(_TA, K, DIM)
    v_lo, v_hi = f32(vw << 16), f32(vw & hi_mask)
    o = jnp.concatenate(
        [jnp.sum(aexp[:, :, 0:HDIM] * v_lo, axis=1),
         jnp.sum(aexp[:, :, HDIM:DIM] * v_hi, axis=1)], axis=1)
    y_ref[...] = jnp.dot(o, wo_ref[...],
                         preferred_element_type=jnp.float32) + bo_ref[...]


def _attn(q, xyzp16, kvnb, xnb, wb1p, bb1_2, wb2T, bb2_2, woT, bo2, bd, bdt):
    bn = q.shape[0]
    grid = (bn // _TA,)
    row = pl.BlockSpec((_TA, DIM), lambda i: (i, 0))
    return pl.pallas_call(
        _attn_body,
        grid=grid,
        in_specs=[
            row,
            pl.BlockSpec((_TA, 128), lambda i: (i, 0)),
            pl.BlockSpec((_TA, K, DIM), lambda i: (i, 0, 0)),
            pl.BlockSpec((_TA, K, 128), lambda i: (i, 0, 0)),
            pl.BlockSpec((128, 32), lambda i: (0, 0)),
            pl.BlockSpec((1, 32), lambda i: (0, 0)),
            pl.BlockSpec((32, H), lambda i: (0, 0)),
            pl.BlockSpec((1, H), lambda i: (0, 0)),
            pl.BlockSpec((DIM, DIM), lambda i: (0, 0)),
            pl.BlockSpec((1, DIM), lambda i: (0, 0)),
            pl.BlockSpec((DIM, H), lambda i: (0, 0)),
            pl.BlockSpec((H, DIM), lambda i: (0, 0)),
        ],
        out_specs=row,
        out_shape=jax.ShapeDtypeStruct((bn, DIM), jnp.float32),
    )(q, xyzp16, kvnb, xnb, wb1p, bb1_2, wb2T, bb2_2, woT, bo2, bd, bdt)


def kernel(x, xyz, Wq, bq, Wk, bk, Wv, bv, Wo, bo, Wb1, bb1, Wb2, bb2):
    bsz, n, c = x.shape
    bn = bsz * n
    xf = x.reshape(bn, c)

    q, kvb = _qkv(xf, Wq.T, Wk.T, Wv.T, bq[None], bk[None], bv[None])

    xyzp8 = jnp.pad(xyz, ((0, 0), (0, 0), (0, 5)))
    xyzT = jnp.swapaxes(xyzp8, 1, 2)
    idx = _knn(xyzp8, xyzT)
    idx_flat = idx.reshape(bn * K)

    xyzp128 = jnp.pad(xyz.reshape(bn, 3), ((0, 0), (0, 125)))
    kvnb, xnb = _make_gather(bn * K)(kvb, xyzp128, idx_flat)

    wb1p = jnp.pad(Wb1.T, ((0, 125), (0, 0)))
    eye = jnp.eye(H, dtype=jnp.float32)
    bd = jnp.repeat(eye, HD, axis=0)
    y = _attn(q, xyzp128,
              kvnb.reshape(bn, K, DIM), xnb.reshape(bn, K, 128),
              wb1p, bb1[None], Wb2.T, bb2[None], Wo.T, bo[None],
              bd, bd.T)
    return y.reshape(bsz, n, c)
